# baseline (speedup 1.0000x reference)
import jax
import jax.numpy as jnp
from jax import lax
from jax.experimental import pallas as pl
from jax.experimental.pallas import tpu as pltpu
from jax.experimental.pallas import tpu_sc as plsc

B = 16384
D = 64
L = 16

_info = plsc.get_sparse_core_info()
NC = _info.num_cores
NS = _info.num_subcores
NW = NC * NS
CHUNK = B // NW
GCH = 128
HALF = CHUNK // 2
NH = HALF // GCH


def _body(uid_hbm, iid_hbm, nid_hbm, utab_hbm, itab_hbm,
          pos_hbm, neg_hbm,
          vid_u, vid_p, vid_n, pid_u, pid_p, pid_n,
          rows_u, rows_p, rows_n, out_p, out_n, sem_idx, sem_g):
    wid = lax.axis_index("s") * NC + lax.axis_index("c")
    base = wid * CHUNK

    idx_dmas = []
    for j in range(CHUNK // GCH):
        off = base + j * GCH
        idx_dmas.append(
            pltpu.async_copy(uid_hbm.at[pl.ds(off, GCH)], vid_u.at[j], sem_idx))
        idx_dmas.append(
            pltpu.async_copy(iid_hbm.at[pl.ds(off, GCH)], vid_p.at[j], sem_idx))
        idx_dmas.append(
            pltpu.async_copy(nid_hbm.at[pl.ds(off, GCH)], vid_n.at[j], sem_idx))
    for h in idx_dmas:
        h.wait()

    lanes = lax.iota(jnp.int32, L)

    for half in range(2):
        hb = half * HALF

        for j in range(NH):
            cj = half * NH + j
            for k in range(GCH // L):
                sl = pl.ds(k * L, L)
                pid_u[j, sl] = vid_u[cj, sl] >> 1
                pid_p[j, sl] = vid_p[cj, sl] >> 1
                pid_n[j, sl] = vid_n[cj, sl] >> 1

        g_dmas = []
        for j in range(NH):
            dst = pl.ds(j * GCH, GCH)
            g_dmas.append(pltpu.async_copy(
                utab_hbm.at[pid_u.at[j]], rows_u.at[dst], sem_g))
            g_dmas.append(pltpu.async_copy(
                itab_hbm.at[pid_p.at[j]], rows_p.at[dst], sem_g))
            g_dmas.append(pltpu.async_copy(
                itab_hbm.at[pid_n.at[j]], rows_n.at[dst], sem_g))
        for h in g_dmas:
            h.wait()

        def group(g, _):
            cj = half * NH + g // (GCH // L)
            ko = (g % (GCH // L)) * L
            par_u = vid_u[cj, pl.ds(ko, L)] & 1
            par_p = vid_p[cj, pl.ds(ko, L)] & 1
            par_n = vid_n[cj, pl.ds(ko, L)] & 1
            accP = jnp.zeros((L,), jnp.float32)
            accN = jnp.zeros((L,), jnp.float32)
            for i in range(L):
                t = g * L + i
                ou = par_u[i] * D
                op = par_p[i] * D
                on = par_n[i] * D
                sp = jnp.zeros((L,), jnp.float32)
                sn = jnp.zeros((L,), jnp.float32)
                for k in range(D // L):
                    u = rows_u[t, pl.ds(ou + k * L, L)]
                    p = rows_p[t, pl.ds(op + k * L, L)]
                    n = rows_n[t, pl.ds(on + k * L, L)]
                    sp = sp + u * p
                    sn = sn + u * n
                accP = jnp.where(lanes == i, jnp.sum(sp), accP)
                accN = jnp.where(lanes == i, jnp.sum(sn), accN)
            out_p[pl.ds(hb + g * L, L)] = accP
            out_n[pl.ds(hb + g * L, L)] = accN
            return _

        lax.fori_loop(0, HALF // L, group, None)

    pltpu.sync_copy(out_p, pos_hbm.at[pl.ds(base, CHUNK)])
    pltpu.sync_copy(out_n, neg_hbm.at[pl.ds(base, CHUNK)])


def kernel(user_id, item_id, neg_item_id, user_table, item_table):
    mesh = plsc.VectorSubcoreMesh(core_axis_name="c", subcore_axis_name="s")
    f = pl.kernel(
        _body,
        mesh=mesh,
        compiler_params=pltpu.CompilerParams(
            needs_layout_passes=False,
        ),
        out_type=(
            jax.ShapeDtypeStruct((B,), jnp.float32),
            jax.ShapeDtypeStruct((B,), jnp.float32),
        ),
        scratch_types=[
            pltpu.VMEM((CHUNK // GCH, GCH), jnp.int32),
            pltpu.VMEM((CHUNK // GCH, GCH), jnp.int32),
            pltpu.VMEM((CHUNK // GCH, GCH), jnp.int32),
            pltpu.VMEM((NH, GCH), jnp.int32),
            pltpu.VMEM((NH, GCH), jnp.int32),
            pltpu.VMEM((NH, GCH), jnp.int32),
            pltpu.VMEM((HALF, 2 * D), jnp.float32),
            pltpu.VMEM((HALF, 2 * D), jnp.float32),
            pltpu.VMEM((HALF, 2 * D), jnp.float32),
            pltpu.VMEM((CHUNK,), jnp.float32),
            pltpu.VMEM((CHUNK,), jnp.float32),
            pltpu.SemaphoreType.DMA,
            pltpu.SemaphoreType.DMA,
        ],
    )
    return f(user_id.astype(jnp.int32), item_id.astype(jnp.int32),
             neg_item_id.astype(jnp.int32),
             user_table.reshape(500000, 2 * D),
             item_table.reshape(500000, 2 * D))
